# baseline (speedup 1.0000x reference)
import functools

import jax
import jax.numpy as jnp
from jax import lax
from jax.experimental import pallas as pl
from jax.experimental.pallas import tpu as pltpu
from jax.experimental.pallas import tpu_sc as plsc

NC = 2
NS = 16
NW = NC * NS

V = 1000000
D = 32
B = 4096 * 200
BPW = B // NW

VB = 128
NBLK = V // VB
VTAIL = V - NBLK * VB
BLK_PER_W = (NBLK + NW - 1) // NW
LINES = V // 4

_mesh = plsc.VectorSubcoreMesh(core_axis_name="c", subcore_axis_name="s")


@functools.partial(
    pl.kernel,
    mesh=_mesh,
    compiler_params=pltpu.CompilerParams(
        use_tc_tiling_on_sc=True, needs_layout_passes=False
    ),
    out_type=jax.ShapeDtypeStruct((LINES, 128), jnp.float32),
    scratch_types=[
        pltpu.VMEM((2, D, VB), jnp.float32),
        pltpu.VMEM((2, D, VB), jnp.float32),
        pltpu.VMEM((D, VTAIL), jnp.float32),
        pltpu.SemaphoreType.DMA,
        pltpu.SemaphoreType.DMA,
        pltpu.SemaphoreType.DMA,
        pltpu.SemaphoreType.DMA,
    ],
)
def _sc_transpose(tT, tail_T, scratch, buf, bufT, tailbuf, ls0, ls1, ws0, ws1):
    wid = lax.axis_index("s") * NC + lax.axis_index("c")
    lsems = (ls0, ls1)
    wsems = (ws0, ws1)
    rows0 = lax.iota(jnp.int32, 16)
    rows1 = rows0 + 16

    def bid(j):
        return jnp.minimum(wid + j * NW, NBLK - 1)

    def start_load(j, s):
        v0 = pl.multiple_of(bid(j) * VB, VB)
        pltpu.async_copy(tT.at[:, pl.ds(v0, VB)], buf.at[s], lsems[s])

    def wait_load(s):
        pltpu.make_async_copy(tT.at[:, pl.ds(0, VB)], buf.at[s], lsems[s]).wait()

    def transpose_into(src, dst, nl):
        @pl.loop(0, nl)
        def _(l):
            for g4 in range(4):
                col = jnp.broadcast_to(l * 4 + g4, (16,)).astype(jnp.int32)
                dst[l, pl.ds(g4 * 32, 16)] = plsc.load_gather(src, [rows0, col])
                dst[l, pl.ds(g4 * 32 + 16, 16)] = plsc.load_gather(src, [rows1, col])

    def start_write(j, s):
        line0 = pl.multiple_of(bid(j) * (VB // 4), 8)
        pltpu.async_copy(bufT.at[s], scratch.at[pl.ds(line0, VB // 4)], wsems[s])

    def wait_write(s):
        pltpu.make_async_copy(
            bufT.at[s], scratch.at[pl.ds(0, VB // 4)], wsems[s]
        ).wait()

    def process(j, s, first):
        wait_load(s)
        if not first:
            wait_write(s)
        transpose_into(buf.at[s], bufT.at[s], D)
        start_write(j, s)

    start_load(0, 0)

    @pl.loop(1, BLK_PER_W - 1, step=2)
    def _(g):
        start_load(g, 1)
        is_first0 = g == 1

        @pl.when(is_first0)
        def _():
            wait_load(0)
            transpose_into(buf.at[0], bufT.at[0], D)
            start_write(g - 1, 0)

        @pl.when(jnp.logical_not(is_first0))
        def _():
            process(g - 1, 0, False)

        start_load(g + 1, 0)
        is_first1 = g == 1

        @pl.when(is_first1)
        def _():
            wait_load(1)
            transpose_into(buf.at[1], bufT.at[1], D)
            start_write(g, 1)

        @pl.when(jnp.logical_not(is_first1))
        def _():
            process(g, 1, False)

    process(BLK_PER_W - 1, 0, False)
    wait_write(1)
    wait_write(0)

    @pl.when(wid == NW - 1)
    def _():
        pltpu.sync_copy(tail_T, tailbuf)
        transpose_into(tailbuf, bufT.at[0], VTAIL // 4)
        pltpu.sync_copy(
            bufT.at[0, pl.ds(0, VTAIL // 4)],
            scratch.at[pl.ds(NBLK * (VB // 4), VTAIL // 4)],
        )


G = 128
K = 8
CHUNK = K * G
NCHUNKS = BPW // CHUNK
IDX_ROWS_PER_W = BPW // G


@functools.partial(
    pl.kernel,
    mesh=_mesh,
    compiler_params=pltpu.CompilerParams(use_tc_tiling_on_sc=False),
    out_type=jax.ShapeDtypeStruct((B, D), jnp.float32),
    scratch_types=[
        pltpu.VMEM((2, K, G), jnp.int32),
        pltpu.VMEM((2, CHUNK, D), jnp.float32),
        pltpu.SemaphoreType.DMA,
        pltpu.SemaphoreType.DMA,
    ],
)
def _sc_gather(idx_hbm, table_hbm, out_hbm, idx_v, rows_v, gsem0, gsem1):
    wid = lax.axis_index("s") * NC + lax.axis_index("c")
    idx_row0 = wid * IDX_ROWS_PER_W
    out_row0 = wid * BPW
    gsems = (gsem0, gsem1)

    def load_idx(j, s):
        pltpu.sync_copy(idx_hbm.at[pl.ds(idx_row0 + j * K, K)], idx_v.at[s])

    def start_gather(s):
        for r in range(K):
            pltpu.async_copy(
                table_hbm.at[idx_v.at[s, r]],
                rows_v.at[s, pl.ds(r * G, G)],
                gsems[s],
            )

    def wait_gather(s):
        pltpu.make_async_copy(
            out_hbm.at[pl.ds(0, CHUNK)], rows_v.at[s], gsems[s]
        ).wait()

    def write_out(j, s):
        pltpu.sync_copy(rows_v.at[s], out_hbm.at[pl.ds(out_row0 + j * CHUNK, CHUNK)])

    load_idx(0, 0)
    start_gather(0)

    @pl.loop(1, NCHUNKS - 1, step=2)
    def _(g):
        load_idx(g, 1)
        start_gather(1)
        wait_gather(0)
        write_out(g - 1, 0)
        load_idx(g + 1, 0)
        start_gather(0)
        wait_gather(1)
        write_out(g, 1)

    wait_gather(0)
    write_out(NCHUNKS - 1, 0)


def kernel(words, word_table):
    tT = word_table.T
    tail_T = word_table[NBLK * VB :, :].T
    scratch = _sc_transpose(tT, tail_T)
    flat_idx = words.reshape(B // G, G)
    out = _sc_gather(flat_idx, scratch.reshape(V, D))
    return out.reshape(*words.shape, D)
